# baseline (speedup 1.0000x reference)
import functools

import jax
import jax.numpy as jnp
from jax import lax
from jax.experimental import pallas as pl
from jax.experimental.pallas import tpu as pltpu
from jax.experimental.pallas import tpu_sc as plsc

_D_MODEL = 4096
_N_EXP = 64
_TOP_K = 8
_ID_BIAS = 1.0
_N_TOKENS = 8192
_NEG_INF = float("-inf")

_NUM_CORES = 2
_NUM_SUBCORES = 16
_LANES = 16
_NW = _NUM_CORES * _NUM_SUBCORES
_TPW = _N_TOKENS // _NW
_GROUPS = _TPW // _LANES


def _proj_block(h_ref, w_ref, out_ref):
    logits = lax.dot_general(
        w_ref[...],
        h_ref[...],
        dimension_numbers=(((1,), (1,)), ((), ())),
        preferred_element_type=jnp.float32,
    )
    idx_col = lax.broadcasted_iota(jnp.int32, (_N_EXP, 128), 0)[:, :1]
    out_ref[...] = logits + jnp.where(idx_col == _N_EXP - 1, _ID_BIAS, 0.0)


@jax.jit
def _proj(h, W):
    block = 1024
    return pl.pallas_call(
        _proj_block,
        grid=(_N_TOKENS // block,),
        in_specs=[
            pl.BlockSpec((block, _D_MODEL), lambda i: (i, 0)),
            pl.BlockSpec((_N_EXP, _D_MODEL), lambda i: (0, 0)),
        ],
        out_specs=pl.BlockSpec((_N_EXP, block), lambda i: (0, i)),
        out_shape=jax.ShapeDtypeStruct((_N_EXP, _N_TOKENS), jnp.float32),
    )(h, W)


_sc_mesh = plsc.VectorSubcoreMesh(
    core_axis_name="c", subcore_axis_name="s"
)


@functools.partial(
    pl.kernel,
    out_type=jax.ShapeDtypeStruct((_N_EXP, _N_TOKENS), jnp.float32),
    mesh=_sc_mesh,
    scratch_types=[
        pltpu.VMEM((_N_EXP, _TPW), jnp.float32),
        pltpu.VMEM((_N_EXP, _TPW), jnp.float32),
    ],
)
def _topk_sc(logits_hbm, gate_hbm, chunk_v, out_v):
    wid = lax.axis_index("s") * _NUM_CORES + lax.axis_index("c")
    base = wid * _TPW
    pltpu.sync_copy(logits_hbm.at[:, pl.ds(base, _TPW)], chunk_v)

    @plsc.parallel_loop(0, _GROUPS)
    def group(g):
        off = g * _LANES
        m = [jnp.full((_LANES,), _NEG_INF, jnp.float32)] * _TOP_K
        for e in range(_N_EXP):
            v = chunk_v[e, pl.ds(off, _LANES)]
            for r in range(_TOP_K):
                hi = jnp.maximum(m[r], v)
                v = jnp.minimum(m[r], v)
                m[r] = hi
        thr = m[_TOP_K - 1]
        cgt = jnp.zeros((_LANES,), jnp.float32)
        for r in range(_TOP_K - 1):
            cgt = cgt + jnp.where(m[r] > thr, 1.0, 0.0)
        need = float(_TOP_K) - cgt
        run = jnp.zeros((_LANES,), jnp.float32)
        for e in range(_N_EXP):
            v = chunk_v[e, pl.ds(off, _LANES)]
            gt_f = jnp.where(v > thr, 1.0, 0.0)
            eq_f = jnp.where(v == thr, 1.0, 0.0)
            ok_f = jnp.where(run < need, eq_f, 0.0)
            run = run + eq_f
            out_v[e, pl.ds(off, _LANES)] = (1.0 / _TOP_K) * (gt_f + ok_f)

    pltpu.sync_copy(out_v, gate_hbm.at[:, pl.ds(base, _TPW)])


@jax.jit
def _router(h, W):
    logits_t = _proj(h, W)
    gate_t = _topk_sc(logits_t)
    gate = gate_t.T
    mask = gate != 0.0
    return mask, gate


def kernel(h, W):
    return _router(h, W)
